# Initial kernel scaffold; baseline (speedup 1.0000x reference)
import jax
import jax.numpy as jnp
from jax.experimental import pallas as pl


def kernel(feat, edge_index, W0_0, b0_0, W0_1, b0_1, ln_g0, ln_b0, W1_0, b1_0, W1_1, b1_1, ln_g1, ln_b1, ws1, ws2):
    raise NotImplementedError("write your pallas kernel here")


# speedup vs baseline: 2.5411x; 2.5411x over previous
import functools

import jax
import jax.numpy as jnp
from jax import lax
from jax.experimental import pallas as pl
from jax.experimental.pallas import tpu as pltpu
from jax.experimental.pallas import tpu_sc as plsc

N = 10000
E = 160000
R = 2
D = 256
DA = 16
HALF = 128

NC = 2
NS = 16
CHUNK = 128
CPT = 80
EPT = CPT * CHUNK
EPAD = NS * EPT
ZROWS = 632
NACC = NS * ZROWS
DUMMY = N
GRP = 16
CW = 16

_PIECES = []
_off = 0
while _off < ZROWS:
    _PIECES.append((_off, min(CHUNK, ZROWS - _off)))
    _off += CHUNK


def _sc_segment_sums(feat_lo, feat_hi, src_r, dst_r):
    mesh = plsc.VectorSubcoreMesh(core_axis_name="c", subcore_axis_name="s")

    @functools.partial(
        pl.kernel,
        mesh=mesh,
        out_type=[
            jax.ShapeDtypeStruct((R, NC, NACC, HALF), jnp.float32),
            jax.ShapeDtypeStruct((R, NACC, HALF), jnp.float32),
        ],
        scratch_types=[
            pltpu.VMEM((GRP, CHUNK), jnp.int32),
            pltpu.VMEM((GRP, CHUNK), jnp.int32),
            pltpu.VMEM((CHUNK, HALF), jnp.float32),
            pltpu.VMEM((16,), jnp.int32),
            pltpu.VMEM_SHARED((NACC, HALF), jnp.float32),
            pltpu.SemaphoreType.DMA,
        ],
    )
    def sc_kernel(feat_lo_h, feat_hi_h, src_h, dst_h,
                  agg_h, cnt_h, idx_s, idx_d, rows, meta,
                  acc, sem):
        cid = lax.axis_index("c")
        t = lax.axis_index("s")

        zero16 = jnp.zeros((16,), jnp.float32)
        one16 = jnp.ones((16,), jnp.float32)

        def fill_rows_zero(i, carry):
            for k in range(HALF // 16):
                rows[i, pl.ds(k * 16, 16)] = zero16
            return carry

        def fill_rows_one(i, carry):
            for k in range(HALF // 16):
                rows[i, pl.ds(k * 16, 16)] = one16
            return carry

        def accumulate(feat_h, c, r):
            base = (r * NS + t) * CPT

            def group(g, carry):
                meta[...] = lax.iota(jnp.int32, 16) + (base + g * GRP)
                pltpu.async_copy(src_h.at[meta], idx_s, sem).wait()
                pltpu.async_copy(dst_h.at[meta], idx_d, sem).wait()

                def chunk(j, carry2):
                    pltpu.async_copy(feat_h.at[idx_s.at[j]], rows,
                                     sem).wait()
                    pltpu.sync_copy(rows, acc.at[idx_d.at[j]], add=True)
                    return carry2

                lax.fori_loop(0, GRP, chunk, 0)
                return carry

            lax.fori_loop(0, CPT // GRP, group, 0)

        for r in range(R):
            lax.fori_loop(0, CHUNK, fill_rows_zero, 0)
            for (off, sz) in _PIECES:
                pltpu.sync_copy(rows.at[pl.ds(0, sz)],
                                acc.at[pl.ds(t * ZROWS + off, sz)])

            plsc.subcore_barrier()

            @pl.when(cid == 0)
            def _():
                accumulate(feat_lo_h, 0, r)

            @pl.when(cid == 1)
            def _():
                accumulate(feat_hi_h, 1, r)

            plsc.subcore_barrier()

            for cc in range(NC):

                @pl.when(cid == cc)
                def _():
                    for (off, sz) in _PIECES:
                        pltpu.sync_copy(acc.at[pl.ds(t * ZROWS + off, sz)],
                                        rows.at[pl.ds(0, sz)])
                        pltpu.sync_copy(
                            rows.at[pl.ds(0, sz)],
                            agg_h.at[r, cc, pl.ds(t * ZROWS + off, sz)])

        lax.fori_loop(0, CHUNK, fill_rows_zero, 0)
        for (off, sz) in _PIECES:
            pltpu.sync_copy(rows.at[pl.ds(0, sz)],
                            acc.at[pl.ds(t * ZROWS + off, sz)])
        lax.fori_loop(0, CHUNK, fill_rows_one, 0)
        plsc.subcore_barrier()

        for r in range(R):

            @pl.when(cid == r)
            def _():
                base = (r * NS + t) * CPT

                def group(g, carry):
                    meta[...] = lax.iota(jnp.int32, 16) + (base + g * GRP)
                    pltpu.async_copy(dst_h.at[meta], idx_d, sem).wait()

                    def chunk(j, carry2):
                        pltpu.sync_copy(rows, acc.at[idx_d.at[j]], add=True)
                        return carry2

                    lax.fori_loop(0, GRP, chunk, 0)
                    return carry

                lax.fori_loop(0, CPT // GRP, group, 0)

        plsc.subcore_barrier()

        for r in range(R):

            @pl.when(cid == r)
            def _():
                for (off, sz) in _PIECES:
                    pltpu.sync_copy(acc.at[pl.ds(t * ZROWS + off, sz)],
                                    rows.at[pl.ds(0, sz)])
                    pltpu.sync_copy(rows.at[pl.ds(0, sz)],
                                    cnt_h.at[r, pl.ds(t * ZROWS + off, sz)])

    return sc_kernel(feat_lo, feat_hi, src_r, dst_r)


def _tc_dense(feat, agg, cnt, params, ws1, ws2r):
    BLK = 1000
    GRID = N // BLK

    def tc_body(feat_ref, agg_ref, cnt_ref,
                W00, b00, W01, b01, g0, be0,
                W10, b10, W11, b11, g1, be1,
                ws1_ref, ws2_ref, out_ref, acc_ref):
        i = pl.program_id(0)
        f = feat_ref[...]
        prm = ((W00, b00, W01, b01, g0, be0), (W10, b10, W11, b11, g1, be1))
        hs, ss = [], []
        for r in range(R):
            a = jnp.concatenate([agg_ref[r, 0], agg_ref[r, 1]], axis=-1)
            c = cnt_ref[r][:, 0:1]
            x = f + a / jnp.maximum(c, 1.0)
            Wa, ba, Wb, bb, g, b = prm[r]
            for (W, bias) in ((Wa, ba), (Wb, bb)):
                x = jnp.dot(x, W[...], preferred_element_type=jnp.float32)
                x = x + bias[...]
                mu = jnp.mean(x, axis=-1, keepdims=True)
                var = jnp.mean((x - mu) ** 2, axis=-1, keepdims=True)
                x = (x - mu) / jnp.sqrt(var + 1e-5) * g[...] + b[...]
                x = jnp.maximum(x, 0.0)
            th = jnp.tanh(jnp.dot(x, ws1_ref[r],
                                  preferred_element_type=jnp.float32))
            s = jnp.sum(th * ws2_ref[r], axis=-1)
            hs.append(x)
            ss.append(s)
        m = jnp.maximum(ss[0], ss[1])
        e0 = jnp.exp(ss[0] - m)
        e1 = jnp.exp(ss[1] - m)
        inv = 1.0 / (e0 + e1)
        hout = hs[0] * (e0 * inv)[:, None] + hs[1] * (e1 * inv)[:, None]
        part = jnp.sum(hout, axis=0, keepdims=True)

        @pl.when(i == 0)
        def _():
            acc_ref[...] = jnp.zeros_like(acc_ref)

        acc_ref[...] += part

        @pl.when(i == GRID - 1)
        def _():
            out_ref[...] = acc_ref[...] * (1.0 / N)

    wspec = pl.BlockSpec((D, D), lambda i: (0, 0))
    vspec = pl.BlockSpec((1, D), lambda i: (0, 0))
    in_specs = [
        pl.BlockSpec((BLK, D), lambda i: (i, 0)),
        pl.BlockSpec((R, NC, BLK, HALF), lambda i: (0, 0, i, 0)),
        pl.BlockSpec((R, BLK, HALF), lambda i: (0, i, 0)),
    ]
    for _ in range(R):
        in_specs += [wspec, vspec, wspec, vspec, vspec, vspec]
    in_specs += [
        pl.BlockSpec((R, D, DA), lambda i: (0, 0, 0)),
        pl.BlockSpec((R, 1, DA), lambda i: (0, 0, 0)),
    ]
    args = [feat, agg, cnt]
    for p in params:
        args += list(p)
    args += [ws1, ws2r]
    return pl.pallas_call(
        tc_body,
        grid=(GRID,),
        in_specs=in_specs,
        out_specs=pl.BlockSpec((1, D), lambda i: (0, 0)),
        out_shape=jax.ShapeDtypeStruct((1, D), jnp.float32),
        scratch_shapes=[pltpu.VMEM((1, D), jnp.float32)],
        compiler_params=pltpu.CompilerParams(
            dimension_semantics=("arbitrary",)),
    )(*args)


def kernel(feat, edge_index, W0_0, b0_0, W0_1, b0_1, ln_g0, ln_b0,
           W1_0, b1_0, W1_1, b1_1, ln_g1, ln_b1, ws1, ws2):
    src = edge_index[:, 0, :].astype(jnp.int32)
    dst = edge_index[:, 1, :].astype(jnp.int32)
    pad = EPAD - E
    src_p = jnp.pad(src, ((0, 0), (0, pad)))
    dst_p = jnp.pad(dst, ((0, 0), (0, pad)), constant_values=DUMMY)
    src_r = src_p.reshape(R * NS * CPT, CHUNK)
    dst_r = dst_p.reshape(R * NS * CPT, CHUNK)
    feat_lo = feat[:, :HALF]
    feat_hi = feat[:, HALF:]

    agg, cnt = _sc_segment_sums(feat_lo, feat_hi, src_r, dst_r)

    params = (
        (W0_0, b0_0.reshape(1, D), W0_1, b0_1.reshape(1, D),
         ln_g0.reshape(1, D), ln_b0.reshape(1, D)),
        (W1_0, b1_0.reshape(1, D), W1_1, b1_1.reshape(1, D),
         ln_g1.reshape(1, D), ln_b1.reshape(1, D)),
    )
    ws2r = ws2.reshape(R, 1, DA)
    return _tc_dense(feat, agg, cnt, params, ws1, ws2r)
